# baseline (speedup 1.0000x reference)
import functools

import jax
import jax.numpy as jnp
from jax import lax
from jax.experimental import pallas as pl
from jax.experimental.pallas import tpu as pltpu
from jax.experimental.pallas import tpu_sc as plsc

EMB = 64
BATCH = 16384
NUM_CORES = 2
NUM_SUBCORES = 16
LANES = 16
VECS = EMB // LANES
NUM_WORKERS = NUM_CORES * NUM_SUBCORES
ROWS_PER_W = BATCH // NUM_WORKERS
IDX_CHUNK = 128
NUM_CHUNKS = ROWS_PER_W // IDX_CHUNK
GROUP = 16


def _decode_body(actions_hbm, table_hbm, state_t_hbm, w4_hbm, w5_hbm, out_hbm,
                 idx_v, za_v, st_v, w4_v, w5_v, s_v, q_v, *sems):
    wid = lax.axis_index("s") * NUM_CORES + lax.axis_index("c")
    base = wid * ROWS_PER_W

    pltpu.sync_copy(actions_hbm.at[pl.ds(wid * NUM_CHUNKS, NUM_CHUNKS)], idx_v)
    copies = []
    for k in range(NUM_CHUNKS):
        copies.append(
            pltpu.async_copy(table_hbm.at[idx_v.at[k]],
                             za_v.at[pl.ds(k * IDX_CHUNK, IDX_CHUNK)], sems[k]))
    st_copy = pltpu.async_copy(
        state_t_hbm.at[:, pl.ds(base, ROWS_PER_W)], st_v, sems[NUM_CHUNKS])
    pltpu.sync_copy(w4_hbm, w4_v)
    pltpu.sync_copy(w5_hbm, w5_v)

    w4vecs = [w4_v[0, pl.ds(t * LANES, LANES)] for t in range(VECS)]
    w5vecs = [w5_v[0, pl.ds(t * LANES, LANES)] for t in range(VECS)]
    w4s = [w4vecs[j // LANES][j % LANES] for j in range(EMB)]
    zero = jnp.zeros((LANES,), jnp.float32)
    lane_iota = lax.iota(jnp.int32, LANES)

    st_copy.wait()

    def s_body(g, carry):
        col = g * LANES
        acc = st_v[0, pl.ds(col, LANES)] * w4s[0]
        for j in range(1, EMB):
            acc = acc + st_v[j, pl.ds(col, LANES)] * w4s[j]
        s_v[pl.ds(col, LANES)] = acc
        return carry

    lax.fori_loop(0, ROWS_PER_W // LANES, s_body, 0)

    for k in range(NUM_CHUNKS):
        copies[k].wait()

        def group_body(g, carry, k=k):
            grow = k * IDX_CHUNK + g * GROUP
            s_vec = s_v[pl.ds(grow, GROUP)]
            q_vec = zero
            for r in range(GROUP):
                row = grow + r
                s_r = s_vec[r]
                qacc = jnp.maximum(za_v[row, pl.ds(0, LANES)] * s_r, 0.0) * w5vecs[0]
                for t in range(1, VECS):
                    qacc = qacc + jnp.maximum(
                        za_v[row, pl.ds(t * LANES, LANES)] * s_r, 0.0) * w5vecs[t]
                q_r = jnp.sum(qacc)
                q_vec = jnp.where(lane_iota == r, q_r, q_vec)
            q_v[pl.ds(grow, GROUP)] = q_vec
            return carry

        lax.fori_loop(0, IDX_CHUNK // GROUP, group_body, 0)

    pltpu.sync_copy(q_v, out_hbm.at[pl.ds(base, ROWS_PER_W)])


@jax.jit
def _decode(actions2d, node_embedding, state_t, w4, w5):
    mesh = plsc.VectorSubcoreMesh(core_axis_name="c", subcore_axis_name="s")
    return pl.kernel(
        _decode_body,
        mesh=mesh,
        compiler_params=pltpu.CompilerParams(
            needs_layout_passes=False, use_tc_tiling_on_sc=False),
        out_type=jax.ShapeDtypeStruct((BATCH,), jnp.float32),
        scratch_types=[
            pltpu.VMEM((NUM_CHUNKS, IDX_CHUNK), jnp.int32),
            pltpu.VMEM((ROWS_PER_W, EMB), jnp.float32),
            pltpu.VMEM((EMB, ROWS_PER_W), jnp.float32),
            pltpu.VMEM((1, EMB), jnp.float32),
            pltpu.VMEM((1, EMB), jnp.float32),
            pltpu.VMEM((ROWS_PER_W,), jnp.float32),
            pltpu.VMEM((ROWS_PER_W,), jnp.float32),
        ] + [pltpu.SemaphoreType.DMA] * (NUM_CHUNKS + 1),
    )(actions2d, node_embedding, state_t, w4, w5)


def kernel(actions, node_embedding, state_embedding, W_4, W_5):
    actions2d = actions.astype(jnp.int32).reshape(BATCH // IDX_CHUNK, IDX_CHUNK)
    out = _decode(actions2d, node_embedding, state_embedding.T,
                  W_4.reshape(1, EMB), W_5.reshape(1, EMB))
    return out.reshape(BATCH, 1)
